# baseline (speedup 1.0000x reference)
import functools

import jax
import jax.numpy as jnp
from jax import lax
from jax.experimental import pallas as pl
from jax.experimental.pallas import tpu as pltpu
from jax.experimental.pallas import tpu_sc as plsc

D_MODEL = 64
SCALE = 8.0

_NC = 2
_NS = 16
_NW = _NC * _NS


@functools.lru_cache(maxsize=None)
def _make_gather(B: int, D: int, C: int):
    b_per_w = B // _NW
    n_chunks = b_per_w // C
    mesh = plsc.VectorSubcoreMesh(core_axis_name="c", subcore_axis_name="s")

    @functools.partial(
        pl.kernel,
        mesh=mesh,
        out_type=jax.ShapeDtypeStruct((B, D), jnp.float32),
        scratch_types=[
            pltpu.VMEM((C,), jnp.int32),
            pltpu.VMEM((C, D), jnp.float32),
            pltpu.SemaphoreType.DMA,
        ],
        compiler_params=pltpu.CompilerParams(use_tc_tiling_on_sc=False),
    )
    def k(x_hbm, table_hbm, out_hbm, idx_v, rows_v, sem):
        wid = lax.axis_index("s") * _NC + lax.axis_index("c")
        base = wid * b_per_w

        def chunk_body(g, carry):
            off = base + g * C
            pltpu.sync_copy(x_hbm.at[pl.ds(off, C)], idx_v)
            pltpu.async_copy(table_hbm.at[idx_v], rows_v, sem).wait()

            def scale_row(i, c):
                for j in range(D // 16):
                    sl = (i, pl.ds(j * 16, 16))
                    rows_v[sl] = rows_v[sl] * SCALE
                return c

            lax.fori_loop(0, C, scale_row, 0)
            pltpu.sync_copy(rows_v, out_hbm.at[pl.ds(off, C)])
            return carry

        lax.fori_loop(0, n_chunks, chunk_body, 0)

    return k


def kernel(x, table):
    B = x.shape[0] * x.shape[1]
    idx = x.reshape(B).astype(jnp.int32)
    out = _make_gather(B, D_MODEL, 512)(idx, table)
    return out.reshape(x.shape[0], x.shape[1], D_MODEL)


# speedup vs baseline: 1.1065x; 1.1065x over previous
import functools

import jax
import jax.numpy as jnp
from jax import lax
from jax.experimental import pallas as pl
from jax.experimental.pallas import tpu as pltpu
from jax.experimental.pallas import tpu_sc as plsc

D_MODEL = 64
SCALE = 8.0

_NC = 2
_NS = 16
_NW = _NC * _NS


@functools.lru_cache(maxsize=None)
def _make_gather(B: int, D: int, C: int):
    b_per_w = B // _NW
    n = b_per_w // C
    assert n % 2 == 0 and n * C == b_per_w and C % 8 == 0
    mesh = plsc.VectorSubcoreMesh(core_axis_name="c", subcore_axis_name="s")

    @functools.partial(
        pl.kernel,
        mesh=mesh,
        out_type=jax.ShapeDtypeStruct((B, D), jnp.float32),
        scratch_types=[
            pltpu.VMEM((C,), jnp.int32),
            pltpu.VMEM((C,), jnp.int32),
            pltpu.VMEM((C, D), jnp.float32),
            pltpu.VMEM((C, D), jnp.float32),
            pltpu.SemaphoreType.DMA,
            pltpu.SemaphoreType.DMA,
            pltpu.SemaphoreType.DMA,
            pltpu.SemaphoreType.DMA,
            pltpu.SemaphoreType.DMA,
            pltpu.SemaphoreType.DMA,
        ],
        compiler_params=pltpu.CompilerParams(use_tc_tiling_on_sc=False),
    )
    def k(x_hbm, table_hbm, out_hbm, i0, i1, r0, r1, si0, si1, sg0, sg1,
          so0, so1):
        idx = (i0, i1)
        rows = (r0, r1)
        sem_i = (si0, si1)
        sem_g = (sg0, sg1)
        sem_o = (so0, so1)
        wid = lax.axis_index("s") * _NC + lax.axis_index("c")
        base = wid * b_per_w

        def start_idx(b, g):
            pltpu.make_async_copy(
                x_hbm.at[pl.ds(base + g * C, C)], idx[b], sem_i[b]).start()

        def wait_idx(b):
            pltpu.make_async_copy(
                x_hbm.at[pl.ds(base, C)], idx[b], sem_i[b]).wait()

        def start_gather(b):
            pltpu.make_async_copy(
                table_hbm.at[idx[b]], rows[b], sem_g[b]).start()

        def wait_gather(b):
            pltpu.make_async_copy(
                table_hbm.at[idx[b]], rows[b], sem_g[b]).wait()

        def start_out(b, g):
            pltpu.make_async_copy(
                rows[b], out_hbm.at[pl.ds(base + g * C, C)], sem_o[b]).start()

        def wait_out(b):
            pltpu.make_async_copy(
                rows[b], out_hbm.at[pl.ds(base, C)], sem_o[b]).wait()

        def scale(b):
            rb = rows[b]

            def body(i, c):
                for j in range(D // 16):
                    sl = (i, pl.ds(j * 16, 16))
                    rb[sl] = rb[sl] * SCALE
                return c

            lax.fori_loop(0, C, body, 0)

        start_idx(0, 0)
        start_idx(1, 1)
        wait_idx(0)
        start_gather(0)

        def outer(o, carry):
            for b in (0, 1):
                g = 2 * o + b
                ob = 1 - b

                @pl.when(g + 1 < n)
                def _():
                    @pl.when(g >= 1)
                    def _():
                        wait_out(ob)

                    wait_idx(ob)
                    start_gather(ob)

                wait_gather(b)

                @pl.when(g + 2 < n)
                def _():
                    start_idx(b, g + 2)

                scale(b)
                start_out(b, g)
            return carry

        lax.fori_loop(0, n // 2, outer, 0)
        wait_out(0)
        wait_out(1)

    return k


def kernel(x, table):
    B = x.shape[0] * x.shape[1]
    idx = x.reshape(B).astype(jnp.int32)
    out = _make_gather(B, D_MODEL, 800)(idx, table)
    return out.reshape(x.shape[0], x.shape[1], D_MODEL)
